# baseline (speedup 1.0000x reference)
import functools

import jax
import jax.numpy as jnp
from jax import lax
from jax.experimental import pallas as pl
from jax.experimental.pallas import tpu as pltpu
from jax.experimental.pallas import tpu_sc as plsc

N = 100000
E = 50000
D = 128
LN_EPS = 1e-3

NC = 2
NS = 16
L = 16

NCHUNKS = 12
CPC = NCHUNKS // NC
CB = 4224
CHUNK_LO = tuple(c * CB for c in range(NCHUNKS))
ACC_ROWS = 4352
DUMP = CB
CSW = CB // NS
LAST_REM = E - (NCHUNKS - 1) * CB
LAST_CSW = 224
LAST_TAIL = LAST_REM - (NS - 1) * LAST_CSW

SCAN_MAIN = 99840
SCAN_PER_TILE = SCAN_MAIN // NS
SCAN_TAIL = N - SCAN_MAIN
SB = 128


@functools.lru_cache(maxsize=1)
def _make_sc_segment_sum():
  mesh = plsc.VectorSubcoreMesh(core_axis_name="c", subcore_axis_name="s",
                                num_cores=NC, num_subcores=NS)

  def body(nodes_hbm, ids_hbm, out_hbm,
           rows_v0, rows_v1, idsv0, idsv1, idx_r, idscan_v, cnt_v, call_v,
           zeros_v, sem_r0, sem_r1, sem_i0, sem_i1, wsem0, wsem1,
           cnt_sh, acc0, acc1):
    rows_bufs = (rows_v0, rows_v1)
    ids_bufs = (idsv0, idsv1)
    sems_r = (sem_r0, sem_r1)
    sems_i = (sem_i0, sem_i1)
    accs = (acc0, acc1)
    wsems = (wsem0, wsem1)
    cid = lax.axis_index("c")
    sid = lax.axis_index("s")

    zvec = jnp.zeros((L,), jnp.float32)

    def _zrow(r, carry):
      for j in range(D // L):
        zeros_v[r, pl.ds(j * L, L)] = zvec
      return carry

    lax.fori_loop(0, zeros_v.shape[0], _zrow, 0)

    base = pl.multiple_of(sid * SCAN_PER_TILE, 8)
    pltpu.sync_copy(ids_hbm.at[pl.ds(base, SCAN_PER_TILE)], idscan_v)

    one = jnp.ones((L,), jnp.int32)
    zero = jnp.zeros((L,), jnp.int32)
    nb = NCHUNKS - 1

    def _count(i, accs):
      v = idscan_v[pl.ds(i * L, L)]
      return tuple(accs[k] + jnp.where(v < CHUNK_LO[k + 1], one, zero)
                   for k in range(nb))

    z = jnp.zeros((L,), jnp.int32)
    cnts = lax.fori_loop(0, SCAN_PER_TILE // L, _count,
                         tuple(z for _ in range(nb)))
    for k in range(nb):
      cnt_v[pl.ds(k * L, L)] = cnts[k]
    pltpu.sync_copy(cnt_v, cnt_sh.at[sid])

    pltpu.sync_copy(ids_hbm.at[pl.ds(SCAN_MAIN, SCAN_TAIL)],
                    idscan_v.at[pl.ds(0, SCAN_TAIL)])

    def _count_tail(i, accs):
      v = idscan_v[pl.ds(i * L, L)]
      return tuple(accs[k] + jnp.where(v < CHUNK_LO[k + 1], one, zero)
                   for k in range(nb))

    tails = lax.fori_loop(0, SCAN_TAIL // L, _count_tail,
                          tuple(z for _ in range(nb)))
    plsc.subcore_barrier()
    pltpu.sync_copy(cnt_sh, call_v)

    sums = list(tails)
    for s in range(NS):
      for k in range(nb):
        sums[k] = sums[k] + call_v[s, pl.ds(k * L, L)]
    rs = [jnp.sum(sums[k]) for k in range(nb)]
    row_lo = tuple([jnp.int32(0)] + rs)
    row_hi = tuple(rs + [jnp.int32(N)])

    iota = lax.iota(jnp.int32, L)
    dump_vec = jnp.full((L,), DUMP, jnp.int32)

    def _wblocks(total):
      return (SB,) * (total // SB) + (
          (total % SB,) if total % SB else ())

    def _strip_sizes(c):
      if CHUNK_LO[c] + CB <= E:
        return CSW, _wblocks(CSW), _wblocks(CSW)
      return LAST_CSW, _wblocks(LAST_CSW), _wblocks(LAST_TAIL)

    def _drain_chunk(c, acc, wsem):
      csw, sizes_main, sizes_last = _strip_sizes(c)
      woff = pl.multiple_of(sid * csw, 8)
      v_lo = CHUNK_LO[c]

      def _drain(sizes):
        wdone = 0
        for n in sizes:
          pltpu.make_async_copy(
              acc.at[pl.ds(woff + wdone, n)],
              out_hbm.at[pl.ds(v_lo + woff + wdone, n)], wsem).wait()
          wdone += n

      @pl.when(sid < NS - 1)
      def _():
        _drain(sizes_main)

      @pl.when(sid == NS - 1)
      def _():
        _drain(sizes_last)

    def do_chunk(c, ci):
      v_lo = CHUNK_LO[c]
      cs = CB
      lo, hi = row_lo[c], row_hi[c]
      p = ci % 2
      acc = accs[p]
      wsem = wsems[p]
      csw, sizes_main, sizes_last = _strip_sizes(c)
      woff = pl.multiple_of(sid * csw, 8)

      def _for_my_sizes(fn):
        @pl.when(sid < NS - 1)
        def _():
          fn(sizes_main)

        @pl.when(sid == NS - 1)
        def _():
          fn(sizes_last)

      if ci >= 2:
        _drain_chunk(c - 2, acc, wsem)

      def _zero(sizes):
        done = 0
        for n in sizes:
          zdone = 0
          while zdone < n:
            zn = min(n - zdone, zeros_v.shape[0])
            pltpu.sync_copy(zeros_v.at[pl.ds(0, zn)],
                            acc.at[pl.ds(woff + done + zdone, zn)])
            zdone += zn
          done += n

      _for_my_sizes(_zero)
      plsc.subcore_barrier()

      lo8 = lo - lax.rem(lo, 8)
      span = hi - lo8
      sub = ((span + 127) // 128) * 8
      a_t = lo8 + sid * sub
      b_t = a_t + sub
      nblk2 = (sub + 2 * SB - 1) // (2 * SB)

      def _start_for(j):
        return pl.multiple_of(jnp.minimum(a_t + j * SB, N - SB), 8)

      def _issue(j, b):
        st = _start_for(j)
        pltpu.async_copy(ids_hbm.at[pl.ds(st, SB)], ids_bufs[b], sems_i[b])
        pltpu.async_copy(nodes_hbm.at[pl.ds(st, SB)], rows_bufs[b],
                         sems_r[b])

      def _wait(b):
        pltpu.make_async_copy(ids_hbm.at[pl.ds(0, SB)], ids_bufs[b],
                              sems_i[b]).wait()
        pltpu.make_async_copy(nodes_hbm.at[pl.ds(0, SB)], rows_bufs[b],
                              sems_r[b]).wait()

      def _process(j, b):
        nominal = a_t + j * SB
        start = _start_for(j)
        for i in range(SB // L):
          v = ids_bufs[b][pl.ds(i * L, L)]
          local = v - v_lo
          rowid = iota + (start + i * L)
          m = ((local >= 0) & (local < cs)
               & (rowid >= nominal) & (rowid < b_t))
          idx = jnp.where(m, local, dump_vec)
          idx_r[0, pl.ds(i * L, L)] = idx
        pltpu.sync_copy(rows_bufs[b], acc.at[idx_r.at[0]], add=True)

      _issue(0, 0)

      def _ring(j2, carry):
        j = 2 * j2
        _wait(0)
        _issue(j + 1, 1)
        _process(j, 0)
        _wait(1)
        _issue(j + 2, 0)
        _process(j + 1, 1)
        return carry

      lax.fori_loop(0, nblk2, _ring, 0)
      _wait(0)
      plsc.subcore_barrier()

      def _write(sizes):
        wdone = 0
        for n in sizes:
          pltpu.async_copy(acc.at[pl.ds(woff + wdone, n)],
                           out_hbm.at[pl.ds(v_lo + woff + wdone, n)],
                           wsem)
          wdone += n

      _for_my_sizes(_write)

    for core in range(NC):
      @pl.when(cid == core)
      def _(core=core):
        chunks = list(range(core * CPC, (core + 1) * CPC))
        for ci, c in enumerate(chunks):
          do_chunk(c, ci)
        for ci, c in enumerate(chunks):
          if ci >= CPC - 2:
            _drain_chunk(c, accs[ci % 2], wsems[ci % 2])

  return pl.kernel(
      body,
      out_type=jax.ShapeDtypeStruct((E, D), jnp.float32),
      mesh=mesh,
      compiler_params=pltpu.CompilerParams(needs_layout_passes=False),
      scratch_types=[
          pltpu.VMEM((SB, D), jnp.float32),
          pltpu.VMEM((SB, D), jnp.float32),
          pltpu.VMEM((SB,), jnp.int32),
          pltpu.VMEM((SB,), jnp.int32),
          pltpu.VMEM((1, 128), jnp.int32),
          pltpu.VMEM((SCAN_PER_TILE,), jnp.int32),
          pltpu.VMEM((256,), jnp.int32),
          pltpu.VMEM((NS, 256), jnp.int32),
          pltpu.VMEM((64, D), jnp.float32),
          pltpu.SemaphoreType.DMA,
          pltpu.SemaphoreType.DMA,
          pltpu.SemaphoreType.DMA,
          pltpu.SemaphoreType.DMA,
          pltpu.SemaphoreType.DMA,
          pltpu.SemaphoreType.DMA,
          pltpu.VMEM_SHARED((NS, 256), jnp.int32),
          pltpu.VMEM_SHARED((ACC_ROWS, D), jnp.float32),
          pltpu.VMEM_SHARED((ACC_ROWS, D), jnp.float32),
      ],
  )


BR = 5000


def _mlp_body(e_ref, a_ref, g_ref, w1_ref, b1_ref, w2_ref, b2_ref,
              gm_ref, bt_ref, o_ref):
  w1 = w1_ref[...]
  x = jnp.dot(e_ref[...], w1[0:D], preferred_element_type=jnp.float32)
  x = x + jnp.dot(a_ref[...], w1[D:2 * D],
                  preferred_element_type=jnp.float32)
  g = jnp.dot(g_ref[...], w1[2 * D:3 * D],
              preferred_element_type=jnp.float32)
  h = jnp.maximum(x + g + b1_ref[...], 0.0)
  h = jnp.maximum(
      jnp.dot(h, w2_ref[...], preferred_element_type=jnp.float32)
      + b2_ref[...], 0.0)
  m = jnp.mean(h, axis=-1, keepdims=True)
  cdev = h - m
  var = jnp.mean(cdev * cdev, axis=-1, keepdims=True)
  o_ref[...] = (cdev * lax.rsqrt(var + LN_EPS)) * gm_ref[...] + bt_ref[...]


def _tc_mlp(edges, agg, globals_, W1, b1, W2, b2, gamma, beta):
  grid = (E // BR,)
  full = lambda shape: pl.BlockSpec(shape, lambda i: (0, 0))
  return pl.pallas_call(
      _mlp_body,
      grid=grid,
      in_specs=[
          pl.BlockSpec((BR, D), lambda i: (i, 0)),
          pl.BlockSpec((BR, D), lambda i: (i, 0)),
          full((1, D)),
          full((3 * D, D)),
          full((1, D)),
          full((D, D)),
          full((1, D)),
          full((1, D)),
          full((1, D)),
      ],
      out_specs=pl.BlockSpec((BR, D), lambda i: (i, 0)),
      out_shape=jax.ShapeDtypeStruct((E, D), jnp.float32),
  )(edges, agg, globals_, W1, b1, W2, b2, gamma, beta)


def kernel(edges, nodes, globals_, segment_ids, num, W1, b1, W2, b2,
           gamma, beta):
  del num
  agg = _make_sc_segment_sum()(nodes, segment_ids)
  row = lambda v: v.reshape(1, D)
  return _tc_mlp(edges, agg, globals_, W1, row(b1), W2, row(b2),
                 row(gamma), row(beta))


# speedup vs baseline: 1.1206x; 1.1206x over previous
import functools

import jax
import jax.numpy as jnp
from jax import lax
from jax.experimental import pallas as pl
from jax.experimental.pallas import tpu as pltpu
from jax.experimental.pallas import tpu_sc as plsc

N = 100000
E = 50000
D = 128
LN_EPS = 1e-3

NC = 2
NS = 16
L = 16

NCHUNKS = 6
CPC = NCHUNKS // NC
CB = 8448
CHUNK_LO = tuple(c * CB for c in range(NCHUNKS))
ACC_ROWS = 8576
DUMP = CB
CSW = CB // NS
LAST_REM = E - (NCHUNKS - 1) * CB
LAST_CSW = 488
LAST_TAIL = LAST_REM - (NS - 1) * LAST_CSW

SCAN_MAIN = 99840
SCAN_PER_TILE = SCAN_MAIN // NS
SCAN_TAIL = N - SCAN_MAIN
SB = 128
NBUF = 3


@functools.lru_cache(maxsize=1)
def _make_sc_segment_sum():
  mesh = plsc.VectorSubcoreMesh(core_axis_name="c", subcore_axis_name="s",
                                num_cores=NC, num_subcores=NS)

  def body(nodes_hbm, ids_hbm, out_hbm,
           rows_v0, rows_v1, rows_v2, idsv0, idsv1, idsv2, idx_r,
           idscan_v, cnt_v, call_v, zeros_v,
           sem_r0, sem_r1, sem_r2, sem_i0, sem_i1, sem_i2,
           cnt_sh, acc):
    rows_bufs = (rows_v0, rows_v1, rows_v2)
    ids_bufs = (idsv0, idsv1, idsv2)
    sems_r = (sem_r0, sem_r1, sem_r2)
    sems_i = (sem_i0, sem_i1, sem_i2)
    cid = lax.axis_index("c")
    sid = lax.axis_index("s")

    zvec = jnp.zeros((L,), jnp.float32)

    def _zrow(r, carry):
      for j in range(D // L):
        zeros_v[r, pl.ds(j * L, L)] = zvec
      return carry

    lax.fori_loop(0, zeros_v.shape[0], _zrow, 0)

    base = pl.multiple_of(sid * SCAN_PER_TILE, 8)
    pltpu.sync_copy(ids_hbm.at[pl.ds(base, SCAN_PER_TILE)], idscan_v)

    one = jnp.ones((L,), jnp.int32)
    zero = jnp.zeros((L,), jnp.int32)
    nb = NCHUNKS - 1

    def _count(i, accs):
      v = idscan_v[pl.ds(i * L, L)]
      return tuple(accs[k] + jnp.where(v < CHUNK_LO[k + 1], one, zero)
                   for k in range(nb))

    z = jnp.zeros((L,), jnp.int32)
    cnts = lax.fori_loop(0, SCAN_PER_TILE // L, _count,
                         tuple(z for _ in range(nb)))
    for k in range(nb):
      cnt_v[pl.ds(k * L, L)] = cnts[k]
    pltpu.sync_copy(cnt_v, cnt_sh.at[sid])

    pltpu.sync_copy(ids_hbm.at[pl.ds(SCAN_MAIN, SCAN_TAIL)],
                    idscan_v.at[pl.ds(0, SCAN_TAIL)])

    def _count_tail(i, accs):
      v = idscan_v[pl.ds(i * L, L)]
      return tuple(accs[k] + jnp.where(v < CHUNK_LO[k + 1], one, zero)
                   for k in range(nb))

    tails = lax.fori_loop(0, SCAN_TAIL // L, _count_tail,
                          tuple(z for _ in range(nb)))
    plsc.subcore_barrier()
    pltpu.sync_copy(cnt_sh, call_v)

    sums = list(tails)
    for s in range(NS):
      for k in range(nb):
        sums[k] = sums[k] + call_v[s, pl.ds(k * L, L)]
    rs = [jnp.sum(sums[k]) for k in range(nb)]
    row_lo = tuple([jnp.int32(0)] + rs)
    row_hi = tuple(rs + [jnp.int32(N)])

    iota = lax.iota(jnp.int32, L)
    dump_vec = jnp.full((L,), DUMP, jnp.int32)

    def _wblocks(total):
      return (SB,) * (total // SB) + (
          (total % SB,) if total % SB else ())

    def _strip_sizes(c):
      if CHUNK_LO[c] + CB <= E:
        return CSW, _wblocks(CSW), _wblocks(CSW)
      return LAST_CSW, _wblocks(LAST_CSW), _wblocks(LAST_TAIL)

    def do_chunk(c):
      v_lo = CHUNK_LO[c]
      cs = CB
      lo, hi = row_lo[c], row_hi[c]
      csw, sizes_main, sizes_last = _strip_sizes(c)
      woff = pl.multiple_of(sid * csw, 8)

      def _for_my_sizes(fn):
        @pl.when(sid < NS - 1)
        def _():
          fn(sizes_main)

        @pl.when(sid == NS - 1)
        def _():
          fn(sizes_last)

      def _zero(sizes):
        done = 0
        for n in sizes:
          zdone = 0
          while zdone < n:
            zn = min(n - zdone, zeros_v.shape[0])
            pltpu.sync_copy(zeros_v.at[pl.ds(0, zn)],
                            acc.at[pl.ds(woff + done + zdone, zn)])
            zdone += zn
          done += n

      _for_my_sizes(_zero)
      plsc.subcore_barrier()

      lo8 = lo - lax.rem(lo, 8)
      span = hi - lo8
      sub = ((span + 127) // 128) * 8
      a_t = lo8 + sid * sub
      b_t = a_t + sub
      nblkr = (sub + NBUF * SB - 1) // (NBUF * SB)

      def _start_for(j):
        return pl.multiple_of(jnp.minimum(a_t + j * SB, N - SB), 8)

      def _issue(j, b):
        st = _start_for(j)
        pltpu.async_copy(ids_hbm.at[pl.ds(st, SB)], ids_bufs[b], sems_i[b])
        pltpu.async_copy(nodes_hbm.at[pl.ds(st, SB)], rows_bufs[b],
                         sems_r[b])

      def _wait(b):
        pltpu.make_async_copy(ids_hbm.at[pl.ds(0, SB)], ids_bufs[b],
                              sems_i[b]).wait()
        pltpu.make_async_copy(nodes_hbm.at[pl.ds(0, SB)], rows_bufs[b],
                              sems_r[b]).wait()

      def _process(j, b):
        nominal = a_t + j * SB
        start = _start_for(j)
        for i in range(SB // L):
          v = ids_bufs[b][pl.ds(i * L, L)]
          local = v - v_lo
          rowid = iota + (start + i * L)
          m = ((local >= 0) & (local < cs)
               & (rowid >= nominal) & (rowid < b_t))
          idx = jnp.where(m, local, dump_vec)
          idx_r[0, pl.ds(i * L, L)] = idx
        pltpu.sync_copy(rows_bufs[b], acc.at[idx_r.at[0]], add=True)

      for b in range(NBUF):
        _issue(b, b)

      def _ring(jr, carry):
        j = NBUF * jr
        for b in range(NBUF):
          _wait(b)
          _process(j + b, b)
          _issue(j + b + NBUF, b)
        return carry

      lax.fori_loop(0, nblkr, _ring, 0)
      for b in range(NBUF):
        _wait(b)
      plsc.subcore_barrier()

      def _write(sizes):
        wdone = 0
        for n in sizes:
          pltpu.sync_copy(acc.at[pl.ds(woff + wdone, n)],
                          out_hbm.at[pl.ds(v_lo + woff + wdone, n)])
          wdone += n

      _for_my_sizes(_write)
      plsc.subcore_barrier()

    for core in range(NC):
      @pl.when(cid == core)
      def _(core=core):
        for c in range(core * CPC, (core + 1) * CPC):
          do_chunk(c)

  return pl.kernel(
      body,
      out_type=jax.ShapeDtypeStruct((E, D), jnp.float32),
      mesh=mesh,
      compiler_params=pltpu.CompilerParams(needs_layout_passes=False),
      scratch_types=[
          pltpu.VMEM((SB, D), jnp.float32),
          pltpu.VMEM((SB, D), jnp.float32),
          pltpu.VMEM((SB, D), jnp.float32),
          pltpu.VMEM((SB,), jnp.int32),
          pltpu.VMEM((SB,), jnp.int32),
          pltpu.VMEM((SB,), jnp.int32),
          pltpu.VMEM((1, 128), jnp.int32),
          pltpu.VMEM((SCAN_PER_TILE,), jnp.int32),
          pltpu.VMEM((128,), jnp.int32),
          pltpu.VMEM((NS, 128), jnp.int32),
          pltpu.VMEM((32, D), jnp.float32),
          pltpu.SemaphoreType.DMA,
          pltpu.SemaphoreType.DMA,
          pltpu.SemaphoreType.DMA,
          pltpu.SemaphoreType.DMA,
          pltpu.SemaphoreType.DMA,
          pltpu.SemaphoreType.DMA,
          pltpu.VMEM_SHARED((NS, 128), jnp.int32),
          pltpu.VMEM_SHARED((ACC_ROWS, D), jnp.float32),
      ],
  )


BR = 5000


def _mlp_body(e_ref, a_ref, g_ref, w1_ref, b1_ref, w2_ref, b2_ref,
              gm_ref, bt_ref, o_ref):
  w1 = w1_ref[...]
  x = jnp.dot(e_ref[...], w1[0:D], preferred_element_type=jnp.float32)
  x = x + jnp.dot(a_ref[...], w1[D:2 * D],
                  preferred_element_type=jnp.float32)
  g = jnp.dot(g_ref[...], w1[2 * D:3 * D],
              preferred_element_type=jnp.float32)
  h = jnp.maximum(x + g + b1_ref[...], 0.0)
  h = jnp.maximum(
      jnp.dot(h, w2_ref[...], preferred_element_type=jnp.float32)
      + b2_ref[...], 0.0)
  m = jnp.mean(h, axis=-1, keepdims=True)
  cdev = h - m
  var = jnp.mean(cdev * cdev, axis=-1, keepdims=True)
  o_ref[...] = (cdev * lax.rsqrt(var + LN_EPS)) * gm_ref[...] + bt_ref[...]


def _tc_mlp(edges, agg, globals_, W1, b1, W2, b2, gamma, beta):
  grid = (E // BR,)
  full = lambda shape: pl.BlockSpec(shape, lambda i: (0, 0))
  return pl.pallas_call(
      _mlp_body,
      grid=grid,
      in_specs=[
          pl.BlockSpec((BR, D), lambda i: (i, 0)),
          pl.BlockSpec((BR, D), lambda i: (i, 0)),
          full((1, D)),
          full((3 * D, D)),
          full((1, D)),
          full((D, D)),
          full((1, D)),
          full((1, D)),
          full((1, D)),
      ],
      out_specs=pl.BlockSpec((BR, D), lambda i: (i, 0)),
      out_shape=jax.ShapeDtypeStruct((E, D), jnp.float32),
  )(edges, agg, globals_, W1, b1, W2, b2, gamma, beta)


def kernel(edges, nodes, globals_, segment_ids, num, W1, b1, W2, b2,
           gamma, beta):
  del num
  agg = _make_sc_segment_sum()(nodes, segment_ids)
  row = lambda v: v.reshape(1, D)
  return _tc_mlp(edges, agg, globals_, W1, row(b1), W2, row(b2),
                 row(gamma), row(beta))


# speedup vs baseline: 1.1212x; 1.0006x over previous
import functools

import jax
import jax.numpy as jnp
from jax import lax
from jax.experimental import pallas as pl
from jax.experimental.pallas import tpu as pltpu
from jax.experimental.pallas import tpu_sc as plsc

N = 100000
E = 50000
D = 128
LN_EPS = 1e-3

NC = 2
NS = 16
L = 16

NCHUNKS = 6
CPC = NCHUNKS // NC
CB = 8448
CHUNK_LO = tuple(c * CB for c in range(NCHUNKS))
ACC_ROWS = 8576
DUMP = CB
CSW = CB // NS
LAST_REM = E - (NCHUNKS - 1) * CB
LAST_CSW = 488
LAST_TAIL = LAST_REM - (NS - 1) * LAST_CSW

SCAN_MAIN = 99840
SCAN_PER_TILE = SCAN_MAIN // NS
SCAN_TAIL = N - SCAN_MAIN
SB = 128
NBUF = 3


@functools.lru_cache(maxsize=1)
def _make_sc_segment_sum():
  mesh = plsc.VectorSubcoreMesh(core_axis_name="c", subcore_axis_name="s",
                                num_cores=NC, num_subcores=NS)

  def body(nodes_hbm, ids_hbm, out_hbm,
           rows_v0, rows_v1, rows_v2, idsv0, idsv1, idsv2, idx_r,
           idscan_v, cnt_v, call_v, zeros_v,
           sem_r0, sem_r1, sem_r2, sem_i0, sem_i1, sem_i2,
           cnt_sh, acc):
    rows_bufs = (rows_v0, rows_v1, rows_v2)
    ids_bufs = (idsv0, idsv1, idsv2)
    sems_r = (sem_r0, sem_r1, sem_r2)
    sems_i = (sem_i0, sem_i1, sem_i2)
    cid = lax.axis_index("c")
    sid = lax.axis_index("s")

    zvec = jnp.zeros((L,), jnp.float32)

    def _zrow(r, carry):
      for j in range(D // L):
        zeros_v[r, pl.ds(j * L, L)] = zvec
      return carry

    lax.fori_loop(0, zeros_v.shape[0], _zrow, 0)

    base = pl.multiple_of(sid * SCAN_PER_TILE, 8)
    pltpu.sync_copy(ids_hbm.at[pl.ds(base, SCAN_PER_TILE)], idscan_v)

    one = jnp.ones((L,), jnp.int32)
    zero = jnp.zeros((L,), jnp.int32)
    nb = NCHUNKS - 1

    def _count(i, accs):
      v = idscan_v[pl.ds(i * L, L)]
      return tuple(accs[k] + jnp.where(v < CHUNK_LO[k + 1], one, zero)
                   for k in range(nb))

    z = jnp.zeros((L,), jnp.int32)
    cnts = lax.fori_loop(0, SCAN_PER_TILE // L, _count,
                         tuple(z for _ in range(nb)))
    for k in range(nb):
      cnt_v[pl.ds(k * L, L)] = cnts[k]
    pltpu.sync_copy(cnt_v, cnt_sh.at[sid])

    pltpu.sync_copy(ids_hbm.at[pl.ds(SCAN_MAIN, SCAN_TAIL)],
                    idscan_v.at[pl.ds(0, SCAN_TAIL)])

    def _count_tail(i, accs):
      v = idscan_v[pl.ds(i * L, L)]
      return tuple(accs[k] + jnp.where(v < CHUNK_LO[k + 1], one, zero)
                   for k in range(nb))

    tails = lax.fori_loop(0, SCAN_TAIL // L, _count_tail,
                          tuple(z for _ in range(nb)))
    plsc.subcore_barrier()
    pltpu.sync_copy(cnt_sh, call_v)

    sums = list(tails)
    for s in range(NS):
      for k in range(nb):
        sums[k] = sums[k] + call_v[s, pl.ds(k * L, L)]
    rs = [jnp.sum(sums[k]) for k in range(nb)]
    row_lo = tuple([jnp.int32(0)] + rs)
    row_hi = tuple(rs + [jnp.int32(N)])

    iota = lax.iota(jnp.int32, L)
    dump_vec = jnp.full((L,), DUMP, jnp.int32)

    def _wblocks(total):
      return (SB,) * (total // SB) + (
          (total % SB,) if total % SB else ())

    def _strip_sizes(c):
      if CHUNK_LO[c] + CB <= E:
        return CSW, _wblocks(CSW), _wblocks(CSW)
      return LAST_CSW, _wblocks(LAST_CSW), _wblocks(LAST_TAIL)

    def do_chunk(c):
      v_lo = CHUNK_LO[c]
      cs = CB
      lo, hi = row_lo[c], row_hi[c]
      csw, sizes_main, sizes_last = _strip_sizes(c)
      woff = pl.multiple_of(sid * csw, 8)

      def _for_my_sizes(fn):
        @pl.when(sid < NS - 1)
        def _():
          fn(sizes_main)

        @pl.when(sid == NS - 1)
        def _():
          fn(sizes_last)

      def _zero(sizes):
        done = 0
        for n in sizes:
          zdone = 0
          while zdone < n:
            zn = min(n - zdone, zeros_v.shape[0])
            pltpu.sync_copy(zeros_v.at[pl.ds(0, zn)],
                            acc.at[pl.ds(woff + done + zdone, zn)])
            zdone += zn
          done += n

      _for_my_sizes(_zero)
      plsc.subcore_barrier()

      lo8 = lo - lax.rem(lo, 8)
      span = hi - lo8
      sub = ((span + 127) // 128) * 8
      a_t = lo8 + sid * sub
      b_t = a_t + sub
      nblkr = (sub + NBUF * SB - 1) // (NBUF * SB)

      def _start_for(j):
        return pl.multiple_of(jnp.minimum(a_t + j * SB, N - SB), 8)

      def _issue(j, b):
        st = _start_for(j)
        pltpu.async_copy(ids_hbm.at[pl.ds(st, SB)], ids_bufs[b], sems_i[b])
        pltpu.async_copy(nodes_hbm.at[pl.ds(st, SB)], rows_bufs[b],
                         sems_r[b])

      def _wait(b):
        pltpu.make_async_copy(ids_hbm.at[pl.ds(0, SB)], ids_bufs[b],
                              sems_i[b]).wait()
        pltpu.make_async_copy(nodes_hbm.at[pl.ds(0, SB)], rows_bufs[b],
                              sems_r[b]).wait()

      def _process(j, b):
        nominal = a_t + j * SB
        start = _start_for(j)
        for i in range(SB // L):
          v = ids_bufs[b][pl.ds(i * L, L)]
          local = v - v_lo
          rowid = iota + (start + i * L)
          m = ((local >= 0) & (local < cs)
               & (rowid >= nominal) & (rowid < b_t))
          idx = jnp.where(m, local, dump_vec)
          idx_r[0, pl.ds(i * L, L)] = idx
        pltpu.sync_copy(rows_bufs[b], acc.at[idx_r.at[0]], add=True)

      for b in range(NBUF):
        _issue(b, b)

      def _ring(jr, carry):
        j = NBUF * jr
        for b in range(NBUF):
          _wait(b)
          _process(j + b, b)
          _issue(j + b + NBUF, b)
        return carry

      lax.fori_loop(0, nblkr, _ring, 0)
      for b in range(NBUF):
        _wait(b)
      plsc.subcore_barrier()

      def _write(sizes):
        wdone = 0
        for n in sizes:
          pltpu.sync_copy(acc.at[pl.ds(woff + wdone, n)],
                          out_hbm.at[pl.ds(v_lo + woff + wdone, n)])
          wdone += n

      _for_my_sizes(_write)

    for core in range(NC):
      @pl.when(cid == core)
      def _(core=core):
        for c in range(core * CPC, (core + 1) * CPC):
          do_chunk(c)

  return pl.kernel(
      body,
      out_type=jax.ShapeDtypeStruct((E, D), jnp.float32),
      mesh=mesh,
      compiler_params=pltpu.CompilerParams(needs_layout_passes=False),
      scratch_types=[
          pltpu.VMEM((SB, D), jnp.float32),
          pltpu.VMEM((SB, D), jnp.float32),
          pltpu.VMEM((SB, D), jnp.float32),
          pltpu.VMEM((SB,), jnp.int32),
          pltpu.VMEM((SB,), jnp.int32),
          pltpu.VMEM((SB,), jnp.int32),
          pltpu.VMEM((1, 128), jnp.int32),
          pltpu.VMEM((SCAN_PER_TILE,), jnp.int32),
          pltpu.VMEM((128,), jnp.int32),
          pltpu.VMEM((NS, 128), jnp.int32),
          pltpu.VMEM((32, D), jnp.float32),
          pltpu.SemaphoreType.DMA,
          pltpu.SemaphoreType.DMA,
          pltpu.SemaphoreType.DMA,
          pltpu.SemaphoreType.DMA,
          pltpu.SemaphoreType.DMA,
          pltpu.SemaphoreType.DMA,
          pltpu.VMEM_SHARED((NS, 128), jnp.int32),
          pltpu.VMEM_SHARED((ACC_ROWS, D), jnp.float32),
      ],
  )


BR = 5000


def _mlp_body(e_ref, a_ref, g_ref, w1_ref, b1_ref, w2_ref, b2_ref,
              gm_ref, bt_ref, o_ref):
  w1 = w1_ref[...]
  x = jnp.dot(e_ref[...], w1[0:D], preferred_element_type=jnp.float32)
  x = x + jnp.dot(a_ref[...], w1[D:2 * D],
                  preferred_element_type=jnp.float32)
  g = jnp.dot(g_ref[...], w1[2 * D:3 * D],
              preferred_element_type=jnp.float32)
  h = jnp.maximum(x + g + b1_ref[...], 0.0)
  h = jnp.maximum(
      jnp.dot(h, w2_ref[...], preferred_element_type=jnp.float32)
      + b2_ref[...], 0.0)
  m = jnp.mean(h, axis=-1, keepdims=True)
  cdev = h - m
  var = jnp.mean(cdev * cdev, axis=-1, keepdims=True)
  o_ref[...] = (cdev * lax.rsqrt(var + LN_EPS)) * gm_ref[...] + bt_ref[...]


def _tc_mlp(edges, agg, globals_, W1, b1, W2, b2, gamma, beta):
  grid = (E // BR,)
  full = lambda shape: pl.BlockSpec(shape, lambda i: (0, 0))
  return pl.pallas_call(
      _mlp_body,
      grid=grid,
      in_specs=[
          pl.BlockSpec((BR, D), lambda i: (i, 0)),
          pl.BlockSpec((BR, D), lambda i: (i, 0)),
          full((1, D)),
          full((3 * D, D)),
          full((1, D)),
          full((D, D)),
          full((1, D)),
          full((1, D)),
          full((1, D)),
      ],
      out_specs=pl.BlockSpec((BR, D), lambda i: (i, 0)),
      out_shape=jax.ShapeDtypeStruct((E, D), jnp.float32),
  )(edges, agg, globals_, W1, b1, W2, b2, gamma, beta)


def kernel(edges, nodes, globals_, segment_ids, num, W1, b1, W2, b2,
           gamma, beta):
  del num
  agg = _make_sc_segment_sum()(nodes, segment_ids)
  row = lambda v: v.reshape(1, D)
  return _tc_mlp(edges, agg, globals_, W1, row(b1), W2, row(b2),
                 row(gamma), row(beta))
